# baseline (speedup 1.0000x reference)
import jax
import jax.numpy as jnp
from jax.experimental import pallas as pl

S = 4096
E = 8
CAP = 1024
ROW_BLOCK = 2048
TB = ROW_BLOCK // E


def _cumsum_rows(x):
    n = x.shape[0]
    k = 1
    while k < n:
        shifted = jnp.concatenate(
            [jnp.zeros((k, x.shape[1]), x.dtype), x[: n - k]], axis=0)
        x = x + shifted
        k *= 2
    return x


def _route_kernel(x_ref, qr_ref, qw_ref):
    x = x_ref[...]
    m = jnp.max(x, axis=-1, keepdims=True)
    ex = jnp.exp(x - m)
    probs = ex / jnp.sum(ex, axis=-1, keepdims=True)

    eio = jax.lax.broadcasted_iota(jnp.int32, (S, E), 1)
    p1 = jnp.max(probs, axis=-1, keepdims=True)
    e1 = jnp.min(jnp.where(probs == p1, eio, E), axis=-1, keepdims=True)
    mask1 = eio == e1
    pe = jnp.where(mask1, -jnp.inf, probs)
    p2 = jnp.max(pe, axis=-1, keepdims=True)
    e2 = jnp.min(jnp.where(pe == p2, eio, E), axis=-1, keepdims=True)
    mask2 = eio == e2

    c1 = _cumsum_rows(mask1.astype(jnp.int32))
    rank1 = c1 - 1
    count1 = c1[S - 1:S, :]
    rank2 = _cumsum_rows(mask2.astype(jnp.int32)) - 1 + count1

    keep1 = mask1 & (rank1 < CAP)
    keep2 = mask2 & (rank2 < CAP)
    slot = jnp.where(keep1, rank1, jnp.where(keep2, rank2, 2 * CAP))
    qr_ref[...] = slot.astype(jnp.float32)
    qw_ref[...] = probs


def _expand_kernel(p_ref, qr_ref, qw_ref, out_ref):
    p = p_ref[...]
    dims = (((1,), (0,)), ((), ()))
    rrq = jax.lax.dot_general(p, qr_ref[...], dims,
                              precision=jax.lax.Precision.HIGHEST,
                              preferred_element_type=jnp.float32)
    rrw = jax.lax.dot_general(p, qw_ref[...], dims,
                              precision=jax.lax.Precision.HIGHEST,
                              preferred_element_type=jnp.float32)
    lane = jax.lax.broadcasted_iota(jnp.int32, (ROW_BLOCK, E), 1)
    row = jax.lax.broadcasted_iota(jnp.int32, (ROW_BLOCK, E), 0)
    lanesel = lane == (row % E)
    qr = jnp.sum(jnp.where(lanesel, rrq, 0.0), axis=1, keepdims=True)
    qw = jnp.sum(jnp.where(lanesel, rrw, 0.0), axis=1, keepdims=True)
    qri = (qr + 0.5).astype(jnp.int32)
    cols = jax.lax.broadcasted_iota(jnp.int32, (ROW_BLOCK, CAP), 1)
    out_ref[...] = jnp.where(cols == qri, qw, 0.0)


def kernel(inputs):
    qr, qw = pl.pallas_call(
        _route_kernel,
        out_shape=(
            jax.ShapeDtypeStruct((S, E), jnp.float32),
            jax.ShapeDtypeStruct((S, E), jnp.float32),
        ),
    )(inputs)

    rows = jnp.arange(ROW_BLOCK, dtype=jnp.int32)[:, None]
    toks = jnp.arange(TB, dtype=jnp.int32)[None, :]
    psel = ((rows // E) == toks).astype(jnp.float32)

    nblk = (S * E) // ROW_BLOCK
    cb = pl.pallas_call(
        _expand_kernel,
        grid=(nblk,),
        in_specs=[
            pl.BlockSpec((ROW_BLOCK, TB), lambda i: (0, 0)),
            pl.BlockSpec((TB, E), lambda i: (i, 0)),
            pl.BlockSpec((TB, E), lambda i: (i, 0)),
        ],
        out_specs=pl.BlockSpec((ROW_BLOCK, CAP), lambda i: (i, 0)),
        out_shape=jax.ShapeDtypeStruct((S * E, CAP), jnp.float32),
    )(psel, qr, qw)

    cio = jnp.arange(CAP, dtype=jnp.int32)
    msk = (qr.astype(jnp.int32)[:, :, None] == cio) & (qw[:, :, None] != 0.0)
    return (cb.reshape(S, E, CAP), msk)


# speedup vs baseline: 1.2325x; 1.2325x over previous
import jax
import jax.numpy as jnp
from jax.experimental import pallas as pl

S = 4096
E = 8
CAP = 1024
ROW_BLOCK = 2048
TB = ROW_BLOCK // E


def _cumsum_rows(x):
    n = x.shape[0]
    k = 1
    while k < n:
        shifted = jnp.concatenate(
            [jnp.zeros((k, x.shape[1]), x.dtype), x[: n - k]], axis=0)
        x = x + shifted
        k *= 2
    return x


def _route_kernel(x_ref, qc_ref):
    x = x_ref[...]
    m = jnp.max(x, axis=-1, keepdims=True)
    ex = jnp.exp(x - m)
    probs = ex / jnp.sum(ex, axis=-1, keepdims=True)

    eio = jax.lax.broadcasted_iota(jnp.int32, (S, E), 1)
    p1 = jnp.max(probs, axis=-1, keepdims=True)
    e1 = jnp.min(jnp.where(probs == p1, eio, E), axis=-1, keepdims=True)
    mask1 = eio == e1
    pe = jnp.where(mask1, -jnp.inf, probs)
    p2 = jnp.max(pe, axis=-1, keepdims=True)
    e2 = jnp.min(jnp.where(pe == p2, eio, E), axis=-1, keepdims=True)
    mask2 = eio == e2

    c1 = _cumsum_rows(mask1.astype(jnp.int32))
    rank1 = c1 - 1
    count1 = c1[S - 1:S, :]
    rank2 = _cumsum_rows(mask2.astype(jnp.int32)) - 1 + count1

    keep1 = mask1 & (rank1 < CAP)
    keep2 = mask2 & (rank2 < CAP)
    slot = jnp.where(keep1, rank1, jnp.where(keep2, rank2, 2 * CAP))

    qa = slot // 64
    qb = slot - qa * 64
    w1 = probs.astype(jnp.bfloat16).astype(jnp.float32)
    r1_ = probs - w1
    w2 = r1_.astype(jnp.bfloat16).astype(jnp.float32)
    w3 = r1_ - w2
    qc_ref[...] = jnp.concatenate(
        [qa.astype(jnp.float32), qb.astype(jnp.float32), w1, w2, w3], axis=1)


def _expand_kernel(p_ref, qc_ref, out_ref):
    dims = (((1,), (0,)), ((), ()))
    rr = jax.lax.dot_general(p_ref[...], qc_ref[...], dims,
                             preferred_element_type=jnp.float32)
    lane = jax.lax.broadcasted_iota(jnp.int32, (ROW_BLOCK, 5 * E), 1)
    row = jax.lax.broadcasted_iota(jnp.int32, (ROW_BLOCK, 5 * E), 0)
    zm = jnp.where((lane % E) == (row % E), rr, 0.0)
    gj = jax.lax.broadcasted_iota(jnp.int32, (5 * E, 5), 0)
    gg = jax.lax.broadcasted_iota(jnp.int32, (5 * E, 5), 1)
    grp = ((gj // E) == gg).astype(jnp.float32)
    d = jax.lax.dot_general(zm, grp, dims,
                            preferred_element_type=jnp.float32)
    slot = d[:, 0:1] * 64.0 + d[:, 1:2]
    qw = (d[:, 2:3] + d[:, 3:4]) + d[:, 4:5]
    qri = (slot + 0.5).astype(jnp.int32)
    cols = jax.lax.broadcasted_iota(jnp.int32, (ROW_BLOCK, CAP), 1)
    out_ref[...] = jnp.where(cols == qri, qw, 0.0)


def kernel(inputs):
    qc = pl.pallas_call(
        _route_kernel,
        out_shape=jax.ShapeDtypeStruct((S, 5 * E), jnp.float32),
    )(inputs)

    rows = jnp.arange(ROW_BLOCK, dtype=jnp.int32)[:, None]
    toks = jnp.arange(TB, dtype=jnp.int32)[None, :]
    psel = ((rows // E) == toks).astype(jnp.float32)

    nblk = (S * E) // ROW_BLOCK
    cb = pl.pallas_call(
        _expand_kernel,
        grid=(nblk,),
        in_specs=[
            pl.BlockSpec((ROW_BLOCK, TB), lambda i: (0, 0)),
            pl.BlockSpec((TB, 5 * E), lambda i: (i, 0)),
        ],
        out_specs=pl.BlockSpec((ROW_BLOCK, CAP), lambda i: (i, 0)),
        out_shape=jax.ShapeDtypeStruct((S * E, CAP), jnp.float32),
    )(psel, qc)

    slot_o = (qc[:, 0:E] * 64.0 + qc[:, E:2 * E]).astype(jnp.int32)
    w_o = (qc[:, 2 * E:3 * E] + qc[:, 3 * E:4 * E]) + qc[:, 4 * E:5 * E]
    cio = jnp.arange(CAP, dtype=jnp.int32)
    msk = (slot_o[:, :, None] == cio) & (w_o[:, :, None] != 0.0)
    return (cb.reshape(S, E, CAP), msk)


# speedup vs baseline: 1.2740x; 1.0337x over previous
import jax
import jax.numpy as jnp
from jax.experimental import pallas as pl

S = 4096
E = 8
CAP = 1024
ROW_BLOCK = 2048
TB = ROW_BLOCK // E


def _cumsum_rows(x):
    n = x.shape[0]
    k = 1
    while k < n:
        shifted = jnp.concatenate(
            [jnp.zeros((k, x.shape[1]), x.dtype), x[: n - k]], axis=0)
        x = x + shifted
        k *= 2
    return x


def _route_kernel(x_ref, qc_ref):
    x = x_ref[...]
    m = jnp.max(x, axis=-1, keepdims=True)
    ex = jnp.exp(x - m)
    probs = ex / jnp.sum(ex, axis=-1, keepdims=True)

    eio = jax.lax.broadcasted_iota(jnp.int32, (S, E), 1)
    p1 = jnp.max(probs, axis=-1, keepdims=True)
    e1 = jnp.min(jnp.where(probs == p1, eio, E), axis=-1, keepdims=True)
    mask1 = eio == e1
    pe = jnp.where(mask1, -jnp.inf, probs)
    p2 = jnp.max(pe, axis=-1, keepdims=True)
    e2 = jnp.min(jnp.where(pe == p2, eio, E), axis=-1, keepdims=True)
    mask2 = eio == e2

    c12 = _cumsum_rows(jnp.concatenate(
        [mask1.astype(jnp.int32), mask2.astype(jnp.int32)], axis=1))
    c1 = c12[:, :E]
    rank1 = c1 - 1
    count1 = c1[S - 1:S, :]
    rank2 = c12[:, E:] - 1 + count1

    keep1 = mask1 & (rank1 < CAP)
    keep2 = mask2 & (rank2 < CAP)
    slot = jnp.where(keep1, rank1, jnp.where(keep2, rank2, 2 * CAP))

    qa = slot // 64
    qb = slot - qa * 64
    w1 = probs.astype(jnp.bfloat16).astype(jnp.float32)
    r1_ = probs - w1
    w2 = r1_.astype(jnp.bfloat16).astype(jnp.float32)
    w3 = r1_ - w2
    qc_ref[...] = jnp.concatenate(
        [qa.astype(jnp.float32), qb.astype(jnp.float32), w1, w2, w3], axis=1)


def _expand_kernel(qc_ref, out_ref):
    pr = jax.lax.broadcasted_iota(jnp.int32, (ROW_BLOCK, TB), 0)
    pt = jax.lax.broadcasted_iota(jnp.int32, (ROW_BLOCK, TB), 1)
    p = ((pr // E) == pt).astype(jnp.float32)
    dims = (((1,), (0,)), ((), ()))
    rr = jax.lax.dot_general(p, qc_ref[...], dims,
                             preferred_element_type=jnp.float32)
    lane = jax.lax.broadcasted_iota(jnp.int32, (ROW_BLOCK, 5 * E), 1)
    row = jax.lax.broadcasted_iota(jnp.int32, (ROW_BLOCK, 5 * E), 0)
    zm = jnp.where((lane % E) == (row % E), rr, 0.0)
    gj = jax.lax.broadcasted_iota(jnp.int32, (5 * E, 5), 0)
    gg = jax.lax.broadcasted_iota(jnp.int32, (5 * E, 5), 1)
    grp = ((gj // E) == gg).astype(jnp.float32)
    d = jax.lax.dot_general(zm, grp, dims,
                            preferred_element_type=jnp.float32)
    slot = d[:, 0:1] * 64.0 + d[:, 1:2]
    qw = (d[:, 2:3] + d[:, 3:4]) + d[:, 4:5]
    qri = (slot + 0.5).astype(jnp.int32)
    cols = jax.lax.broadcasted_iota(jnp.int32, (ROW_BLOCK, CAP), 1)
    out_ref[...] = jnp.where(cols == qri, qw, 0.0)


def kernel(inputs):
    qc = pl.pallas_call(
        _route_kernel,
        out_shape=jax.ShapeDtypeStruct((S, 5 * E), jnp.float32),
    )(inputs)

    nblk = (S * E) // ROW_BLOCK
    cb = pl.pallas_call(
        _expand_kernel,
        grid=(nblk,),
        in_specs=[
            pl.BlockSpec((TB, 5 * E), lambda i: (i, 0)),
        ],
        out_specs=pl.BlockSpec((ROW_BLOCK, CAP), lambda i: (i, 0)),
        out_shape=jax.ShapeDtypeStruct((S * E, CAP), jnp.float32),
    )(qc)

    slot_o = (qc[:, 0:E] * 64.0 + qc[:, E:2 * E]).astype(jnp.int32)
    w_o = (qc[:, 2 * E:3 * E] + qc[:, 3 * E:4 * E]) + qc[:, 4 * E:5 * E]
    ms = jnp.where(w_o != 0.0, slot_o, -1)
    cio = jnp.arange(CAP, dtype=jnp.int32)
    msk = ms[:, :, None] == cio
    return (cb.reshape(S, E, CAP), msk)
